# Initial kernel scaffold; baseline (speedup 1.0000x reference)
import jax
import jax.numpy as jnp
from jax.experimental import pallas as pl


def kernel(x, edge_index, batch, W1, b1, W2, b2, Wc, bc):
    raise NotImplementedError("write your pallas kernel here")


# speedup vs baseline: 34.2355x; 34.2355x over previous
import functools

import jax
import jax.numpy as jnp
from jax import lax
from jax.experimental import pallas as pl
from jax.experimental.pallas import tpu as pltpu
from jax.experimental.pallas import tpu_sc as plsc

N_NODES = 10000
NP = 10240
E_EDGES = 320000
F_IN = 128
H = 64
G = 64
C = 2

NC, NS = 2, 16
NW = NC * NS
E_PER_TILE = 10240
CHUNK = 128
NCHUNK = E_PER_TILE // CHUNK
EP = NW * E_PER_TILE
ROWS_PER_TILE = NP // NS
DEG_W = 16
UNROLL = 8

BR = 512
NB = NP // BR

def _mesh():
    return plsc.VectorSubcoreMesh(core_axis_name="c", subcore_axis_name="s",
                                  num_cores=NC, num_subcores=NS)


def _sc_deg(dst3, zeros16, ones_chunk):
    @functools.partial(
        pl.kernel,
        out_type=jax.ShapeDtypeStruct((NC, NP, DEG_W), jnp.float32),
        mesh=_mesh(),
        compiler_params=pltpu.CompilerParams(use_tc_tiling_on_sc=False),
        scratch_types=[
            pltpu.VMEM((NCHUNK, CHUNK), jnp.int32),
            pltpu.VMEM((CHUNK, DEG_W), jnp.float32),
            pltpu.VMEM_SHARED((NP, DEG_W), jnp.float32),
        ],
    )
    def k(dst_hbm, z_hbm, ones_hbm, out_hbm, idx_v, ones_v, acc):
        cid = lax.axis_index("c")
        sid = lax.axis_index("s")
        wid = cid * NS + sid
        pltpu.sync_copy(dst_hbm.at[wid], idx_v)
        pltpu.sync_copy(ones_hbm, ones_v)
        r0 = sid * ROWS_PER_TILE
        pltpu.sync_copy(z_hbm.at[pl.ds(r0, ROWS_PER_TILE)],
                        acc.at[pl.ds(r0, ROWS_PER_TILE)])
        plsc.subcore_barrier()

        def body(j, carry):
            pltpu.sync_copy(ones_v, acc.at[idx_v.at[j]], add=True)
            return carry

        lax.fori_loop(0, NCHUNK, body, 0)
        plsc.subcore_barrier()
        pltpu.sync_copy(acc.at[pl.ds(r0, ROWS_PER_TILE)],
                        out_hbm.at[cid, pl.ds(r0, ROWS_PER_TILE)])

    return k(dst3, zeros16, ones_chunk)


def _sc_scatter(y, src3, dst3, zeros64):
    @functools.partial(
        pl.kernel,
        out_type=jax.ShapeDtypeStruct((NC, NP, H), jnp.float32),
        mesh=_mesh(),
        compiler_params=pltpu.CompilerParams(use_tc_tiling_on_sc=False),
        scratch_types=[
            pltpu.VMEM((NCHUNK, CHUNK), jnp.int32),
            pltpu.VMEM((NCHUNK, CHUNK), jnp.int32),
            pltpu.VMEM((CHUNK, H), jnp.float32),
            pltpu.VMEM((CHUNK, H), jnp.float32),
            pltpu.SemaphoreType.DMA,
            pltpu.SemaphoreType.DMA,
            pltpu.VMEM_SHARED((NP, H), jnp.float32),
        ],
    )
    def k(y_hbm, src_hbm, dst_hbm, z_hbm, out_hbm,
          src_v, dst_v, buf_a, buf_b, sem_a, sem_b, acc):
        cid = lax.axis_index("c")
        sid = lax.axis_index("s")
        wid = cid * NS + sid
        pltpu.sync_copy(src_hbm.at[wid], src_v)
        pltpu.sync_copy(dst_hbm.at[wid], dst_v)
        r0 = sid * ROWS_PER_TILE
        pltpu.sync_copy(z_hbm.at[pl.ds(r0, ROWS_PER_TILE)],
                        acc.at[pl.ds(r0, ROWS_PER_TILE)])
        plsc.subcore_barrier()

        bufs = (buf_a, buf_b)
        sems = (sem_a, sem_b)

        def outer(t, carry):
            base = t * UNROLL
            d = pltpu.async_copy(y_hbm.at[src_v.at[base]], buf_a, sem_a)
            for kk in range(UNROLL):
                j = base + kk
                if kk + 1 < UNROLL:
                    d_next = pltpu.async_copy(
                        y_hbm.at[src_v.at[j + 1]],
                        bufs[(kk + 1) % 2], sems[(kk + 1) % 2])
                d.wait()
                pltpu.sync_copy(bufs[kk % 2], acc.at[dst_v.at[j]], add=True)
                if kk + 1 < UNROLL:
                    d = d_next
            return carry

        lax.fori_loop(0, NCHUNK // UNROLL, outer, 0)
        plsc.subcore_barrier()
        pltpu.sync_copy(acc.at[pl.ds(r0, ROWS_PER_TILE)],
                        out_hbm.at[cid, pl.ds(r0, ROWS_PER_TILE)])

    return k(y, src3, dst3, zeros64)


def _tc1(x_p, W1, degp):
    def body(x_ref, w_ref, d_ref, o_ref):
        deg = d_ref[0] + d_ref[1]
        dinv = lax.rsqrt(deg[:, :1] + 1.0)
        h = jnp.dot(x_ref[...], w_ref[...], preferred_element_type=jnp.float32)
        i = pl.program_id(0)
        rows = lax.broadcasted_iota(jnp.int32, (BR, 1), 0) + i * BR
        o_ref[...] = jnp.where(rows < N_NODES, h * dinv, 0.0)

    return pl.pallas_call(
        body,
        grid=(NB,),
        in_specs=[
            pl.BlockSpec((BR, F_IN), lambda i: (i, 0)),
            pl.BlockSpec((F_IN, H), lambda i: (0, 0)),
            pl.BlockSpec((NC, BR, DEG_W), lambda i: (0, i, 0)),
        ],
        out_specs=pl.BlockSpec((BR, H), lambda i: (i, 0)),
        out_shape=jax.ShapeDtypeStruct((NP, H), jnp.float32),
    )(x_p, W1, degp)


def _tc2(p, y1, degp, b1r, W2):
    def body(p_ref, y_ref, d_ref, b_ref, w_ref, o_ref):
        deg = d_ref[0] + d_ref[1]
        dinv = lax.rsqrt(deg[:, :1] + 1.0)
        s = (p_ref[0] + p_ref[1] + y_ref[...]) * dinv + b_ref[...]
        h = jnp.maximum(s, 0.0)
        y2 = jnp.dot(h, w_ref[...], preferred_element_type=jnp.float32) * dinv
        i = pl.program_id(0)
        rows = lax.broadcasted_iota(jnp.int32, (BR, 1), 0) + i * BR
        o_ref[...] = jnp.where(rows < N_NODES, y2, 0.0)

    return pl.pallas_call(
        body,
        grid=(NB,),
        in_specs=[
            pl.BlockSpec((NC, BR, H), lambda i: (0, i, 0)),
            pl.BlockSpec((BR, H), lambda i: (i, 0)),
            pl.BlockSpec((NC, BR, DEG_W), lambda i: (0, i, 0)),
            pl.BlockSpec((1, H), lambda i: (0, 0)),
            pl.BlockSpec((H, H), lambda i: (0, 0)),
        ],
        out_specs=pl.BlockSpec((BR, H), lambda i: (i, 0)),
        out_shape=jax.ShapeDtypeStruct((NP, H), jnp.float32),
    )(p, y1, degp, b1r, W2)


def _tc3(q, y2, degp, b2r, batch2d, Wc, bcr):
    def body(q_ref, y_ref, d_ref, b_ref, bt_ref, wc_ref, bc_ref, o_ref,
             s_acc, c_acc):
        i = pl.program_id(0)

        @pl.when(i == 0)
        def _():
            s_acc[...] = jnp.zeros_like(s_acc)
            c_acc[...] = jnp.zeros_like(c_acc)

        deg = d_ref[0] + d_ref[1]
        dinv = lax.rsqrt(deg[:, :1] + 1.0)
        s = (q_ref[0] + q_ref[1] + y_ref[...]) * dinv + b_ref[...]
        h = jnp.maximum(s, 0.0)
        bt = bt_ref[...]
        gi = lax.broadcasted_iota(jnp.int32, (G, BR), 0)
        onehot = (gi == bt).astype(jnp.float32)
        s_acc[...] += jnp.dot(onehot, h, preferred_element_type=jnp.float32)
        c_acc[...] += jnp.sum(onehot, axis=1, keepdims=True)

        @pl.when(i == NB - 1)
        def _():
            pooled = s_acc[...] / jnp.maximum(c_acc[...], 1.0)
            o_ref[...] = (jnp.dot(pooled, wc_ref[...],
                                  preferred_element_type=jnp.float32)
                          + bc_ref[...])

    return pl.pallas_call(
        body,
        grid=(NB,),
        in_specs=[
            pl.BlockSpec((NC, BR, H), lambda i: (0, i, 0)),
            pl.BlockSpec((BR, H), lambda i: (i, 0)),
            pl.BlockSpec((NC, BR, DEG_W), lambda i: (0, i, 0)),
            pl.BlockSpec((1, H), lambda i: (0, 0)),
            pl.BlockSpec((1, BR), lambda i: (0, i)),
            pl.BlockSpec((H, C), lambda i: (0, 0)),
            pl.BlockSpec((1, C), lambda i: (0, 0)),
        ],
        out_specs=pl.BlockSpec((G, C), lambda i: (0, 0)),
        out_shape=jax.ShapeDtypeStruct((G, C), jnp.float32),
        scratch_shapes=[
            pltpu.VMEM((G, H), jnp.float32),
            pltpu.VMEM((G, 1), jnp.float32),
        ],
    )(q, y2, degp, b2r, batch2d, Wc, bcr)


def kernel(x, edge_index, batch, W1, b1, W2, b2, Wc, bc):
    x_p = jnp.zeros((NP, F_IN), x.dtype).at[:N_NODES].set(x)
    pad = (jnp.arange(EP - E_EDGES, dtype=jnp.int32) % (NP - N_NODES)
           + N_NODES)
    src3 = jnp.concatenate([edge_index[0], pad]).reshape(NW, NCHUNK, CHUNK)
    dst3 = jnp.concatenate([edge_index[1], pad]).reshape(NW, NCHUNK, CHUNK)
    zeros16 = jnp.zeros((NP, DEG_W), jnp.float32)
    zeros64 = jnp.zeros((NP, H), jnp.float32)
    ones_chunk = jnp.ones((CHUNK, DEG_W), jnp.float32)
    batch2d = jnp.full((1, NP), -1, jnp.int32).at[0, :N_NODES].set(batch)

    degp = _sc_deg(dst3, zeros16, ones_chunk)
    y1 = _tc1(x_p, W1, degp)
    p1 = _sc_scatter(y1, src3, dst3, zeros64)
    y2 = _tc2(p1, y1, degp, b1.reshape(1, H), W2)
    p2 = _sc_scatter(y2, src3, dst3, zeros64)
    logits = _tc3(p2, y2, degp, b2.reshape(1, H), batch2d, Wc,
                  bc.reshape(1, C))
    return logits


# speedup vs baseline: 54.4254x; 1.5897x over previous
import functools

import jax
import jax.numpy as jnp
from jax import lax
from jax.experimental import pallas as pl
from jax.experimental.pallas import tpu as pltpu
from jax.experimental.pallas import tpu_sc as plsc

N_NODES = 10000
NP = 10240
E_EDGES = 320000
F_IN = 128
H = 64
G = 64
C = 2

NC, NS = 2, 16
NW = NC * NS
CHUNK = 128
NCH_TOT = E_EDGES // CHUNK
NCH_BASE = NCH_TOT // NW
NEXTRA = NCH_TOT - NCH_BASE * NW
ROWS_PER_TILE = NP // NS
DEG_W = 16
KG = 3
NGROUP = NCH_BASE // (2 * KG)

BR = NP
NB = NP // BR

def _mesh():
    return plsc.VectorSubcoreMesh(core_axis_name="c", subcore_axis_name="s",
                                  num_cores=NC, num_subcores=NS)


def _tile_chunks(cid, sid):
    wid = cid * NS + sid
    has_extra = wid < NEXTRA
    base_el = (NCH_BASE * wid + jnp.minimum(wid, NEXTRA)) * CHUNK
    return wid, has_extra, base_el


def _sc_deg(ei, zeros16, ones_chunk):
    @functools.partial(
        pl.kernel,
        out_type=jax.ShapeDtypeStruct((NC, NP, DEG_W), jnp.float32),
        mesh=_mesh(),
        compiler_params=pltpu.CompilerParams(use_tc_tiling_on_sc=False),
        scratch_types=[
            pltpu.VMEM((NCH_BASE + 1, CHUNK), jnp.int32),
            pltpu.VMEM((CHUNK, DEG_W), jnp.float32),
            pltpu.SemaphoreType.DMA,
            pltpu.VMEM_SHARED((NP, DEG_W), jnp.float32),
        ],
    )
    def k(ei_hbm, z_hbm, ones_hbm, out_hbm, idx_v, ones_v, sem, acc):
        cid = lax.axis_index("c")
        sid = lax.axis_index("s")
        _, has_extra, base_el = _tile_chunks(cid, sid)
        dd = [pltpu.async_copy(
            ei_hbm.at[1, pl.ds(base_el + j * CHUNK, CHUNK)], idx_v.at[j], sem)
            for j in range(NCH_BASE)]
        pltpu.sync_copy(ones_hbm, ones_v)
        r0 = sid * ROWS_PER_TILE
        pltpu.sync_copy(z_hbm.at[pl.ds(r0, ROWS_PER_TILE)],
                        acc.at[pl.ds(r0, ROWS_PER_TILE)])

        @pl.when(has_extra)
        def _():
            pltpu.sync_copy(ei_hbm.at[1, pl.ds(base_el + NCH_BASE * CHUNK,
                                               CHUNK)], idx_v.at[NCH_BASE])

        for d in dd:
            d.wait()
        plsc.subcore_barrier()

        def body(j, carry):
            pltpu.async_copy(ones_v, acc.at[idx_v.at[j]], sem, add=True)
            return carry

        lax.fori_loop(0, NCH_BASE, body, 0)

        @pl.when(has_extra)
        def _():
            pltpu.sync_copy(ones_v, acc.at[idx_v.at[NCH_BASE]], add=True)

        def drain(j, carry):
            pltpu.make_async_copy(ones_v, acc.at[idx_v.at[0]], sem).wait()
            return carry

        lax.fori_loop(0, NCH_BASE, drain, 0)
        plsc.subcore_barrier()
        pltpu.sync_copy(acc.at[pl.ds(r0, ROWS_PER_TILE)],
                        out_hbm.at[cid, pl.ds(r0, ROWS_PER_TILE)])

    return k(ei, zeros16, ones_chunk)


def _sc_scatter(y, ei, zeros64):
    @functools.partial(
        pl.kernel,
        out_type=jax.ShapeDtypeStruct((NC, NP, H), jnp.bfloat16),
        mesh=_mesh(),
        compiler_params=pltpu.CompilerParams(use_tc_tiling_on_sc=False),
        scratch_types=[
            pltpu.VMEM(((NCH_BASE + 1) * CHUNK,), jnp.int32),
            pltpu.VMEM((NCH_BASE + 1, CHUNK), jnp.int32),
            pltpu.VMEM((2 * KG * CHUNK, H), jnp.bfloat16),
            pltpu.SemaphoreType.DMA,
            pltpu.SemaphoreType.DMA,
            pltpu.SemaphoreType.DMA,
            pltpu.SemaphoreType.DMA,
            pltpu.VMEM_SHARED((NP, H), jnp.bfloat16),
        ],
    )
    def k(y_hbm, ei_hbm, z_hbm, out_hbm,
          src_v, dst_v, buf, sem_ag, sem_bg, sem_as, sem_bs, acc):
        cid = lax.axis_index("c")
        sid = lax.axis_index("s")
        _, has_extra, base_el = _tile_chunks(cid, sid)
        dsrc = pltpu.async_copy(
            ei_hbm.at[0, pl.ds(base_el, NCH_BASE * CHUNK)],
            src_v.at[pl.ds(0, NCH_BASE * CHUNK)], sem_ag)
        dd = [pltpu.async_copy(
            ei_hbm.at[1, pl.ds(base_el + j * CHUNK, CHUNK)], dst_v.at[j],
            sem_bg) for j in range(NCH_BASE)]
        r0 = sid * ROWS_PER_TILE
        pltpu.sync_copy(z_hbm.at[pl.ds(r0, ROWS_PER_TILE)],
                        acc.at[pl.ds(r0, ROWS_PER_TILE)])

        @pl.when(has_extra)
        def _():
            pltpu.sync_copy(
                ei_hbm.at[0, pl.ds(base_el + NCH_BASE * CHUNK, CHUNK)],
                src_v.at[pl.ds(NCH_BASE * CHUNK, CHUNK)])
            pltpu.sync_copy(
                ei_hbm.at[1, pl.ds(base_el + NCH_BASE * CHUNK, CHUNK)],
                dst_v.at[NCH_BASE])

        dsrc.wait()
        for d in dd:
            d.wait()
        plsc.subcore_barrier()

        def bufsl(i):
            return buf.at[pl.ds(i * CHUNK, CHUNK)]

        def gather(j, i, sem):
            return pltpu.async_copy(
                y_hbm.at[src_v.at[pl.ds(j * CHUNK, CHUNK)]], bufsl(i), sem)

        def scat(j, i, sem):
            return pltpu.async_copy(bufsl(i), acc.at[dst_v.at[j]], sem,
                                    add=True)

        def drain_gather(i, sem):
            pltpu.make_async_copy(
                y_hbm.at[src_v.at[pl.ds(0, CHUNK)]], bufsl(i), sem).wait()

        for kk in range(KG):
            gather(kk, kk, sem_ag)

        def body(tt, carry):
            base = tt * 2 * KG
            db = [gather(base + KG + kk, KG + kk, sem_bg) for kk in range(KG)]
            for kk in range(KG):
                drain_gather(kk, sem_ag)
            da = [scat(base + kk, kk, sem_as) for kk in range(KG)]
            for d in db:
                d.wait()
            for d in da:
                d.wait()

            @pl.when(tt < NGROUP - 1)
            def _():
                for kk in range(KG):
                    gather(base + 2 * KG + kk, kk, sem_ag)

            dbs = [scat(base + KG + kk, KG + kk, sem_bs) for kk in range(KG)]
            for d in dbs:
                d.wait()
            return carry

        lax.fori_loop(0, NGROUP, body, 0)

        @pl.when(has_extra)
        def _():
            pltpu.sync_copy(
                y_hbm.at[src_v.at[pl.ds(NCH_BASE * CHUNK, CHUNK)]], bufsl(0))
            pltpu.sync_copy(bufsl(0), acc.at[dst_v.at[NCH_BASE]], add=True)

        plsc.subcore_barrier()
        pltpu.sync_copy(acc.at[pl.ds(r0, ROWS_PER_TILE)],
                        out_hbm.at[cid, pl.ds(r0, ROWS_PER_TILE)])

    return k(y, ei, zeros64)


def _tc1(x, W1, degp):
    def body(x_ref, w_ref, d_ref, o_ref, ob_ref):
        deg = d_ref[0] + d_ref[1]
        dinv = lax.rsqrt(deg[:, :1] + 1.0)
        h = jnp.dot(x_ref[...], w_ref[...], preferred_element_type=jnp.float32)
        i = pl.program_id(0)
        rows = lax.broadcasted_iota(jnp.int32, (BR, 1), 0) + i * BR
        y = jnp.where(rows < N_NODES, h * dinv, 0.0)
        o_ref[...] = y
        ob_ref[...] = y.astype(jnp.bfloat16)

    return pl.pallas_call(
        body,
        grid=(NB,),
        in_specs=[
            pl.BlockSpec((BR, F_IN), lambda i: (i, 0)),
            pl.BlockSpec((F_IN, H), lambda i: (0, 0)),
            pl.BlockSpec((NC, BR, DEG_W), lambda i: (0, i, 0)),
        ],
        out_specs=[
            pl.BlockSpec((BR, H), lambda i: (i, 0)),
            pl.BlockSpec((BR, H), lambda i: (i, 0)),
        ],
        out_shape=[
            jax.ShapeDtypeStruct((NP, H), jnp.float32),
            jax.ShapeDtypeStruct((NP, H), jnp.bfloat16),
        ],
    )(x, W1, degp)


def _tc2(p, y1, degp, b1r, W2):
    def body(p_ref, y_ref, d_ref, b_ref, w_ref, o_ref, ob_ref):
        deg = d_ref[0] + d_ref[1]
        dinv = lax.rsqrt(deg[:, :1] + 1.0)
        p = (p_ref[0].astype(jnp.float32) + p_ref[1].astype(jnp.float32)
             + y_ref[...])
        s = p * dinv + b_ref[...]
        h = jnp.maximum(s, 0.0)
        y2 = jnp.dot(h, w_ref[...], preferred_element_type=jnp.float32) * dinv
        i = pl.program_id(0)
        rows = lax.broadcasted_iota(jnp.int32, (BR, 1), 0) + i * BR
        y2 = jnp.where(rows < N_NODES, y2, 0.0)
        o_ref[...] = y2
        ob_ref[...] = y2.astype(jnp.bfloat16)

    return pl.pallas_call(
        body,
        grid=(NB,),
        in_specs=[
            pl.BlockSpec((NC, BR, H), lambda i: (0, i, 0)),
            pl.BlockSpec((BR, H), lambda i: (i, 0)),
            pl.BlockSpec((NC, BR, DEG_W), lambda i: (0, i, 0)),
            pl.BlockSpec((1, H), lambda i: (0, 0)),
            pl.BlockSpec((H, H), lambda i: (0, 0)),
        ],
        out_specs=[
            pl.BlockSpec((BR, H), lambda i: (i, 0)),
            pl.BlockSpec((BR, H), lambda i: (i, 0)),
        ],
        out_shape=[
            jax.ShapeDtypeStruct((NP, H), jnp.float32),
            jax.ShapeDtypeStruct((NP, H), jnp.bfloat16),
        ],
    )(p, y1, degp, b1r, W2)


def _tc3(q, y2, degp, b2r, batch2d, Wc, bcr):
    def body(q_ref, y_ref, d_ref, b_ref, bt_ref, wc_ref, bc_ref, o_ref,
             s_acc, c_acc):
        i = pl.program_id(0)

        @pl.when(i == 0)
        def _():
            s_acc[...] = jnp.zeros_like(s_acc)
            c_acc[...] = jnp.zeros_like(c_acc)

        deg = d_ref[0] + d_ref[1]
        dinv = lax.rsqrt(deg[:, :1] + 1.0)
        q = (q_ref[0].astype(jnp.float32) + q_ref[1].astype(jnp.float32)
             + y_ref[...])
        s = q * dinv + b_ref[...]
        h = jnp.maximum(s, 0.0)
        bt = bt_ref[...]
        gi = lax.broadcasted_iota(jnp.int32, (G, BR), 0)
        onehot = (gi == bt).astype(jnp.float32)
        s_acc[...] += jnp.dot(onehot, h, preferred_element_type=jnp.float32)
        c_acc[...] += jnp.sum(onehot, axis=1, keepdims=True)

        @pl.when(i == NB - 1)
        def _():
            pooled = s_acc[...] / jnp.maximum(c_acc[...], 1.0)
            o_ref[...] = (jnp.dot(pooled, wc_ref[...],
                                  preferred_element_type=jnp.float32)
                          + bc_ref[...])

    return pl.pallas_call(
        body,
        grid=(NB,),
        in_specs=[
            pl.BlockSpec((NC, BR, H), lambda i: (0, i, 0)),
            pl.BlockSpec((BR, H), lambda i: (i, 0)),
            pl.BlockSpec((NC, BR, DEG_W), lambda i: (0, i, 0)),
            pl.BlockSpec((1, H), lambda i: (0, 0)),
            pl.BlockSpec((1, BR), lambda i: (0, i)),
            pl.BlockSpec((H, C), lambda i: (0, 0)),
            pl.BlockSpec((1, C), lambda i: (0, 0)),
        ],
        out_specs=pl.BlockSpec((G, C), lambda i: (0, 0)),
        out_shape=jax.ShapeDtypeStruct((G, C), jnp.float32),
        scratch_shapes=[
            pltpu.VMEM((G, H), jnp.float32),
            pltpu.VMEM((G, 1), jnp.float32),
        ],
    )(q, y2, degp, b2r, batch2d, Wc, bcr)


def kernel(x, edge_index, batch, W1, b1, W2, b2, Wc, bc):
    zeros16 = jnp.zeros((NP, DEG_W), jnp.float32)
    zeros64 = jnp.zeros((NP, H), jnp.bfloat16)
    ones_chunk = jnp.ones((CHUNK, DEG_W), jnp.float32)
    batch2d = jnp.full((1, NP), -1, jnp.int32).at[0, :N_NODES].set(batch)

    degp = _sc_deg(edge_index, zeros16, ones_chunk)
    y1, y1b = _tc1(x, W1, degp)
    p1 = _sc_scatter(y1b, edge_index, zeros64)
    y2, y2b = _tc2(p1, y1, degp, b1.reshape(1, H), W2)
    p2 = _sc_scatter(y2b, edge_index, zeros64)
    logits = _tc3(p2, y2, degp, b2.reshape(1, H), batch2d, Wc,
                  bc.reshape(1, C))
    return logits


# speedup vs baseline: 55.4320x; 1.0185x over previous
import functools

import jax
import jax.numpy as jnp
from jax import lax
from jax.experimental import pallas as pl
from jax.experimental.pallas import tpu as pltpu
from jax.experimental.pallas import tpu_sc as plsc

N_NODES = 10000
NP = 10240
E_EDGES = 320000
F_IN = 128
H = 64
G = 64
C = 2

NC, NS = 2, 16
NW = NC * NS
CHUNK = 128
NCH_TOT = E_EDGES // CHUNK
NCH_BASE = NCH_TOT // NW
NEXTRA = NCH_TOT - NCH_BASE * NW
ROWS_PER_TILE = NP // NS
DEG_W = 8
KG = 3
NGROUP = NCH_BASE // (2 * KG)

BR = NP
NB = NP // BR

def _mesh():
    return plsc.VectorSubcoreMesh(core_axis_name="c", subcore_axis_name="s",
                                  num_cores=NC, num_subcores=NS)


def _tile_chunks(cid, sid):
    wid = cid * NS + sid
    has_extra = wid < NEXTRA
    base_el = (NCH_BASE * wid + jnp.minimum(wid, NEXTRA)) * CHUNK
    return wid, has_extra, base_el


def _sc_deg(ei, zeros16, ones_chunk):
    @functools.partial(
        pl.kernel,
        out_type=jax.ShapeDtypeStruct((NC, NP, DEG_W), jnp.float32),
        mesh=_mesh(),
        compiler_params=pltpu.CompilerParams(use_tc_tiling_on_sc=False),
        scratch_types=[
            pltpu.VMEM((NCH_BASE + 1, CHUNK), jnp.int32),
            pltpu.VMEM((CHUNK, DEG_W), jnp.float32),
            pltpu.SemaphoreType.DMA,
            pltpu.VMEM_SHARED((NP, DEG_W), jnp.float32),
        ],
    )
    def k(ei_hbm, z_hbm, ones_hbm, out_hbm, idx_v, ones_v, sem, acc):
        cid = lax.axis_index("c")
        sid = lax.axis_index("s")
        _, has_extra, base_el = _tile_chunks(cid, sid)
        dd = [pltpu.async_copy(
            ei_hbm.at[1, pl.ds(base_el + j * CHUNK, CHUNK)], idx_v.at[j], sem)
            for j in range(NCH_BASE)]
        pltpu.sync_copy(ones_hbm, ones_v)
        r0 = sid * ROWS_PER_TILE
        pltpu.sync_copy(z_hbm.at[pl.ds(r0, ROWS_PER_TILE)],
                        acc.at[pl.ds(r0, ROWS_PER_TILE)])

        @pl.when(has_extra)
        def _():
            pltpu.sync_copy(ei_hbm.at[1, pl.ds(base_el + NCH_BASE * CHUNK,
                                               CHUNK)], idx_v.at[NCH_BASE])

        for d in dd:
            d.wait()
        plsc.subcore_barrier()

        def body(j, carry):
            pltpu.async_copy(ones_v, acc.at[idx_v.at[j]], sem, add=True)
            return carry

        lax.fori_loop(0, NCH_BASE, body, 0)

        @pl.when(has_extra)
        def _():
            pltpu.sync_copy(ones_v, acc.at[idx_v.at[NCH_BASE]], add=True)

        def drain(j, carry):
            pltpu.make_async_copy(ones_v, acc.at[idx_v.at[0]], sem).wait()
            return carry

        lax.fori_loop(0, NCH_BASE, drain, 0)
        plsc.subcore_barrier()
        pltpu.sync_copy(acc.at[pl.ds(r0, ROWS_PER_TILE)],
                        out_hbm.at[cid, pl.ds(r0, ROWS_PER_TILE)])

    return k(ei, zeros16, ones_chunk)


def _sc_scatter(y, ei, zeros64):
    @functools.partial(
        pl.kernel,
        out_type=jax.ShapeDtypeStruct((NC, NP, H), jnp.bfloat16),
        mesh=_mesh(),
        compiler_params=pltpu.CompilerParams(use_tc_tiling_on_sc=False),
        scratch_types=[
            pltpu.VMEM(((NCH_BASE + 1) * CHUNK,), jnp.int32),
            pltpu.VMEM((NCH_BASE + 1, CHUNK), jnp.int32),
            pltpu.VMEM((2 * KG * CHUNK, H), jnp.bfloat16),
            pltpu.SemaphoreType.DMA,
            pltpu.SemaphoreType.DMA,
            pltpu.SemaphoreType.DMA,
            pltpu.SemaphoreType.DMA,
            pltpu.VMEM_SHARED((NP, H), jnp.bfloat16),
        ],
    )
    def k(y_hbm, ei_hbm, z_hbm, out_hbm,
          src_v, dst_v, buf, sem_ag, sem_bg, sem_as, sem_bs, acc):
        cid = lax.axis_index("c")
        sid = lax.axis_index("s")
        _, has_extra, base_el = _tile_chunks(cid, sid)
        dsrc = pltpu.async_copy(
            ei_hbm.at[0, pl.ds(base_el, NCH_BASE * CHUNK)],
            src_v.at[pl.ds(0, NCH_BASE * CHUNK)], sem_ag)
        dd = [pltpu.async_copy(
            ei_hbm.at[1, pl.ds(base_el + j * CHUNK, CHUNK)], dst_v.at[j],
            sem_bg) for j in range(NCH_BASE)]
        r0 = sid * ROWS_PER_TILE
        pltpu.sync_copy(z_hbm.at[pl.ds(r0, ROWS_PER_TILE)],
                        acc.at[pl.ds(r0, ROWS_PER_TILE)])

        @pl.when(has_extra)
        def _():
            pltpu.sync_copy(
                ei_hbm.at[0, pl.ds(base_el + NCH_BASE * CHUNK, CHUNK)],
                src_v.at[pl.ds(NCH_BASE * CHUNK, CHUNK)])
            pltpu.sync_copy(
                ei_hbm.at[1, pl.ds(base_el + NCH_BASE * CHUNK, CHUNK)],
                dst_v.at[NCH_BASE])

        dsrc.wait()
        for d in dd:
            d.wait()
        plsc.subcore_barrier()

        def bufsl(i):
            return buf.at[pl.ds(i * CHUNK, CHUNK)]

        def gather(j, i, sem):
            return pltpu.async_copy(
                y_hbm.at[src_v.at[pl.ds(j * CHUNK, CHUNK)]], bufsl(i), sem)

        def scat(j, i, sem):
            return pltpu.async_copy(bufsl(i), acc.at[dst_v.at[j]], sem,
                                    add=True)

        def drain_gather(i, sem):
            pltpu.make_async_copy(
                y_hbm.at[src_v.at[pl.ds(0, CHUNK)]], bufsl(i), sem).wait()

        for kk in range(KG):
            gather(kk, kk, sem_ag)

        def body(tt, carry):
            base = tt * 2 * KG
            db = [gather(base + KG + kk, KG + kk, sem_bg) for kk in range(KG)]
            for kk in range(KG):
                drain_gather(kk, sem_ag)
            da = [scat(base + kk, kk, sem_as) for kk in range(KG)]
            for d in db:
                d.wait()
            for d in da:
                d.wait()

            @pl.when(tt < NGROUP - 1)
            def _():
                for kk in range(KG):
                    gather(base + 2 * KG + kk, kk, sem_ag)

            dbs = [scat(base + KG + kk, KG + kk, sem_bs) for kk in range(KG)]
            for d in dbs:
                d.wait()
            return carry

        lax.fori_loop(0, NGROUP, body, 0)

        @pl.when(has_extra)
        def _():
            pltpu.sync_copy(
                y_hbm.at[src_v.at[pl.ds(NCH_BASE * CHUNK, CHUNK)]], bufsl(0))
            pltpu.sync_copy(bufsl(0), acc.at[dst_v.at[NCH_BASE]], add=True)

        plsc.subcore_barrier()
        pltpu.sync_copy(acc.at[pl.ds(r0, ROWS_PER_TILE)],
                        out_hbm.at[cid, pl.ds(r0, ROWS_PER_TILE)])

    return k(y, ei, zeros64)


def _tc1(x, W1, degp):
    def body(x_ref, w_ref, d_ref, o_ref, ob_ref):
        deg = d_ref[0] + d_ref[1]
        dinv = lax.rsqrt(deg[:, :1] + 1.0)
        h = jnp.dot(x_ref[...], w_ref[...], preferred_element_type=jnp.float32)
        i = pl.program_id(0)
        rows = lax.broadcasted_iota(jnp.int32, (BR, 1), 0) + i * BR
        y = jnp.where(rows < N_NODES, h * dinv, 0.0)
        o_ref[...] = y
        ob_ref[...] = y.astype(jnp.bfloat16)

    return pl.pallas_call(
        body,
        grid=(NB,),
        in_specs=[
            pl.BlockSpec((BR, F_IN), lambda i: (i, 0)),
            pl.BlockSpec((F_IN, H), lambda i: (0, 0)),
            pl.BlockSpec((NC, BR, DEG_W), lambda i: (0, i, 0)),
        ],
        out_specs=[
            pl.BlockSpec((BR, H), lambda i: (i, 0)),
            pl.BlockSpec((BR, H), lambda i: (i, 0)),
        ],
        out_shape=[
            jax.ShapeDtypeStruct((NP, H), jnp.float32),
            jax.ShapeDtypeStruct((NP, H), jnp.bfloat16),
        ],
    )(x, W1, degp)


def _tc2(p, y1, degp, b1r, W2):
    def body(p_ref, y_ref, d_ref, b_ref, w_ref, o_ref, ob_ref):
        deg = d_ref[0] + d_ref[1]
        dinv = lax.rsqrt(deg[:, :1] + 1.0)
        p = (p_ref[0].astype(jnp.float32) + p_ref[1].astype(jnp.float32)
             + y_ref[...])
        s = p * dinv + b_ref[...]
        h = jnp.maximum(s, 0.0)
        y2 = jnp.dot(h, w_ref[...], preferred_element_type=jnp.float32) * dinv
        i = pl.program_id(0)
        rows = lax.broadcasted_iota(jnp.int32, (BR, 1), 0) + i * BR
        y2 = jnp.where(rows < N_NODES, y2, 0.0)
        o_ref[...] = y2
        ob_ref[...] = y2.astype(jnp.bfloat16)

    return pl.pallas_call(
        body,
        grid=(NB,),
        in_specs=[
            pl.BlockSpec((NC, BR, H), lambda i: (0, i, 0)),
            pl.BlockSpec((BR, H), lambda i: (i, 0)),
            pl.BlockSpec((NC, BR, DEG_W), lambda i: (0, i, 0)),
            pl.BlockSpec((1, H), lambda i: (0, 0)),
            pl.BlockSpec((H, H), lambda i: (0, 0)),
        ],
        out_specs=[
            pl.BlockSpec((BR, H), lambda i: (i, 0)),
            pl.BlockSpec((BR, H), lambda i: (i, 0)),
        ],
        out_shape=[
            jax.ShapeDtypeStruct((NP, H), jnp.float32),
            jax.ShapeDtypeStruct((NP, H), jnp.bfloat16),
        ],
    )(p, y1, degp, b1r, W2)


def _tc3(q, y2, degp, b2r, batch2d, Wc, bcr):
    def body(q_ref, y_ref, d_ref, b_ref, bt_ref, wc_ref, bc_ref, o_ref,
             s_acc, c_acc):
        i = pl.program_id(0)

        @pl.when(i == 0)
        def _():
            s_acc[...] = jnp.zeros_like(s_acc)
            c_acc[...] = jnp.zeros_like(c_acc)

        deg = d_ref[0] + d_ref[1]
        dinv = lax.rsqrt(deg[:, :1] + 1.0)
        q = (q_ref[0].astype(jnp.float32) + q_ref[1].astype(jnp.float32)
             + y_ref[...])
        s = q * dinv + b_ref[...]
        h = jnp.maximum(s, 0.0)
        bt = bt_ref[...]
        gi = lax.broadcasted_iota(jnp.int32, (G, BR), 0)
        onehot = (gi == bt).astype(jnp.float32)
        s_acc[...] += jnp.dot(onehot, h, preferred_element_type=jnp.float32)
        c_acc[...] += jnp.sum(onehot, axis=1, keepdims=True)

        @pl.when(i == NB - 1)
        def _():
            pooled = s_acc[...] / jnp.maximum(c_acc[...], 1.0)
            o_ref[...] = (jnp.dot(pooled, wc_ref[...],
                                  preferred_element_type=jnp.float32)
                          + bc_ref[...])

    return pl.pallas_call(
        body,
        grid=(NB,),
        in_specs=[
            pl.BlockSpec((NC, BR, H), lambda i: (0, i, 0)),
            pl.BlockSpec((BR, H), lambda i: (i, 0)),
            pl.BlockSpec((NC, BR, DEG_W), lambda i: (0, i, 0)),
            pl.BlockSpec((1, H), lambda i: (0, 0)),
            pl.BlockSpec((1, BR), lambda i: (0, i)),
            pl.BlockSpec((H, C), lambda i: (0, 0)),
            pl.BlockSpec((1, C), lambda i: (0, 0)),
        ],
        out_specs=pl.BlockSpec((G, C), lambda i: (0, 0)),
        out_shape=jax.ShapeDtypeStruct((G, C), jnp.float32),
        scratch_shapes=[
            pltpu.VMEM((G, H), jnp.float32),
            pltpu.VMEM((G, 1), jnp.float32),
        ],
    )(q, y2, degp, b2r, batch2d, Wc, bcr)


def kernel(x, edge_index, batch, W1, b1, W2, b2, Wc, bc):
    zeros16 = jnp.zeros((NP, DEG_W), jnp.float32)
    zeros64 = jnp.zeros((NP, H), jnp.bfloat16)
    ones_chunk = jnp.ones((CHUNK, DEG_W), jnp.float32)
    batch2d = jnp.full((1, NP), -1, jnp.int32).at[0, :N_NODES].set(batch)

    degp = _sc_deg(edge_index, zeros16, ones_chunk)
    y1, y1b = _tc1(x, W1, degp)
    p1 = _sc_scatter(y1b, edge_index, zeros64)
    y2, y2b = _tc2(p1, y1, degp, b1.reshape(1, H), W2)
    p2 = _sc_scatter(y2b, edge_index, zeros64)
    logits = _tc3(p2, y2, degp, b2.reshape(1, H), batch2d, Wc,
                  bc.reshape(1, C))
    return logits
